# baseline (speedup 1.0000x reference)
import functools

import jax
import jax.numpy as jnp
from jax.experimental import pallas as pl
from jax.experimental.pallas import tpu as pltpu

H = 2048
E_DIM = 1024
N_SHARED = 2
N_ROUTED = 8
K = 2
T = 4096

LANES = 128
TBLK = 256


def _router_body(x_ref, w_ref, gate_ref):
    logits = jax.lax.dot_general(
        x_ref[...], w_ref[...], (((1,), (0,)), ((), ())),
        preferred_element_type=jnp.float32)
    lane = jax.lax.broadcasted_iota(jnp.int32, logits.shape, 1)
    valid = lane < N_ROUTED
    neg = jnp.full_like(logits, -jnp.inf)
    l = jnp.where(valid, logits, neg)
    m1 = jnp.max(l, axis=-1, keepdims=True)
    i1 = jnp.min(jnp.where(l == m1, lane, N_ROUTED + 7), axis=-1, keepdims=True)
    l2 = jnp.where(lane == i1, neg, l)
    m2 = jnp.max(l2, axis=-1, keepdims=True)
    i2 = jnp.min(jnp.where(l2 == m2, lane, N_ROUTED + 7), axis=-1, keepdims=True)
    z = jnp.sum(jnp.where(valid, jnp.exp(l - m1), 0.0), axis=-1, keepdims=True)
    p1 = 1.0 / z
    p2 = jnp.exp(m2 - m1) / z
    gate = jnp.where(lane == i1, p1, 0.0) + jnp.where(lane == i2, p2, 0.0)
    gate_ref[...] = jnp.where(valid, gate, 0.0)


def _router_gate(x, router_w):
    w_pad = jnp.zeros((H, LANES), jnp.float32).at[:, :N_ROUTED].set(router_w.T)
    gate = pl.pallas_call(
        _router_body,
        grid=(T // TBLK,),
        in_specs=[
            pl.BlockSpec((TBLK, H), lambda t: (t, 0)),
            pl.BlockSpec((H, LANES), lambda t: (0, 0)),
        ],
        out_specs=pl.BlockSpec((TBLK, LANES), lambda t: (t, 0)),
        out_shape=jax.ShapeDtypeStruct((T, LANES), jnp.float32),
    )(x, w_pad)
    return gate


def _moe_body(g_ref, x_ref, up_ref, down_ref, out_ref):
    e = pl.program_id(1)
    x = x_ref[...]
    up = up_ref[0]
    down = down_ref[0]
    h = jax.lax.dot_general(x, up, (((1,), (1,)), ((), ())),
                            preferred_element_type=jnp.float32)
    h = h * 0.5 * (1.0 + jax.lax.erf(h * 0.7071067811865476))
    y = jax.lax.dot_general(h, down, (((1,), (1,)), ((), ())),
                            preferred_element_type=jnp.float32)
    y = y * g_ref[0, 0, :][:, None]

    @pl.when(e == 0)
    def _():
        out_ref[...] = y

    @pl.when(e > 0)
    def _():
        out_ref[...] += y


def kernel(x, shared_up, shared_down, routed_up, routed_down, router_w):
    gate = _router_gate(x, router_w)
    g_routed = gate[:, :N_ROUTED].T
    g_all = jnp.concatenate(
        [jnp.ones((N_SHARED, T), jnp.float32), g_routed], axis=0)
    g_all = g_all[:, None, :]
    up_all = jnp.concatenate([shared_up, routed_up], axis=0)
    down_all = jnp.concatenate([shared_down, routed_down], axis=0)
    ne = N_SHARED + N_ROUTED

    out = pl.pallas_call(
        _moe_body,
        grid=(T // TBLK, ne),
        in_specs=[
            pl.BlockSpec((1, 1, TBLK), lambda t, e: (e, 0, t)),
            pl.BlockSpec((TBLK, H), lambda t, e: (t, 0)),
            pl.BlockSpec((1, E_DIM, H), lambda t, e: (e, 0, 0)),
            pl.BlockSpec((1, H, E_DIM), lambda t, e: (e, 0, 0)),
        ],
        out_specs=pl.BlockSpec((TBLK, H), lambda t, e: (t, 0)),
        out_shape=jax.ShapeDtypeStruct((T, H), jnp.float32),
    )(g_all, x, up_all, down_all)
    return out


# speedup vs baseline: 1.3639x; 1.3639x over previous
import functools

import jax
import jax.numpy as jnp
from jax.experimental import pallas as pl
from jax.experimental.pallas import tpu as pltpu

H = 2048
E_DIM = 1024
N_SHARED = 2
N_ROUTED = 8
K = 2
T = 4096

LANES = 128
TBLK = 256


def _router_body(x_ref, w_ref, gate_ref):
    logits = jax.lax.dot_general(
        x_ref[...], w_ref[...], (((1,), (0,)), ((), ())),
        preferred_element_type=jnp.float32)
    lane = jax.lax.broadcasted_iota(jnp.int32, logits.shape, 1)
    valid = lane < N_ROUTED
    neg = jnp.full_like(logits, -jnp.inf)
    l = jnp.where(valid, logits, neg)
    m1 = jnp.max(l, axis=-1, keepdims=True)
    i1 = jnp.min(jnp.where(l == m1, lane, N_ROUTED + 7), axis=-1, keepdims=True)
    l2 = jnp.where(lane == i1, neg, l)
    m2 = jnp.max(l2, axis=-1, keepdims=True)
    i2 = jnp.min(jnp.where(l2 == m2, lane, N_ROUTED + 7), axis=-1, keepdims=True)
    z = jnp.sum(jnp.where(valid, jnp.exp(l - m1), 0.0), axis=-1, keepdims=True)
    p1 = 1.0 / z
    p2 = jnp.exp(m2 - m1) / z
    gate = jnp.where(lane == i1, p1, 0.0) + jnp.where(lane == i2, p2, 0.0)
    gate_ref[...] = jnp.where(valid, gate, 0.0)


def _router_gate(x, router_w):
    w_pad = jnp.zeros((H, LANES), jnp.float32).at[:, :N_ROUTED].set(router_w.T)
    gate = pl.pallas_call(
        _router_body,
        grid=(T // TBLK,),
        in_specs=[
            pl.BlockSpec((TBLK, H), lambda t: (t, 0)),
            pl.BlockSpec((H, LANES), lambda t: (0, 0)),
        ],
        out_specs=pl.BlockSpec((TBLK, LANES), lambda t: (t, 0)),
        out_shape=jax.ShapeDtypeStruct((T, LANES), jnp.float32),
    )(x, w_pad)
    return gate


def _moe_body(g_ref, x_ref, up_ref, down_ref, out_ref):
    e = pl.program_id(1)
    x = x_ref[...]
    up = up_ref[0]
    down = down_ref[0]
    h = jax.lax.dot_general(x, up, (((1,), (1,)), ((), ())),
                            preferred_element_type=jnp.float32)
    h = h * 0.5 * (1.0 + jax.lax.erf(h * 0.7071067811865476))
    y = jax.lax.dot_general(h.astype(jnp.bfloat16), down, (((1,), (1,)), ((), ())),
                            preferred_element_type=jnp.float32)
    y = y * g_ref[0, 0, :][:, None]

    @pl.when(e == 0)
    def _():
        out_ref[...] = y

    @pl.when(e > 0)
    def _():
        out_ref[...] += y


def kernel(x, shared_up, shared_down, routed_up, routed_down, router_w):
    gate = _router_gate(x, router_w)
    g_routed = gate[:, :N_ROUTED].T
    g_all = jnp.concatenate(
        [jnp.ones((N_SHARED, T), jnp.float32), g_routed], axis=0)
    g_all = g_all[:, None, :]
    up_all = jnp.concatenate([shared_up, routed_up], axis=0).astype(jnp.bfloat16)
    down_all = jnp.concatenate([shared_down, routed_down], axis=0).astype(jnp.bfloat16)
    x_bf = x.astype(jnp.bfloat16)
    ne = N_SHARED + N_ROUTED

    out = pl.pallas_call(
        _moe_body,
        grid=(T // TBLK, ne),
        in_specs=[
            pl.BlockSpec((1, 1, TBLK), lambda t, e: (e, 0, t)),
            pl.BlockSpec((TBLK, H), lambda t, e: (t, 0)),
            pl.BlockSpec((1, E_DIM, H), lambda t, e: (e, 0, 0)),
            pl.BlockSpec((1, H, E_DIM), lambda t, e: (e, 0, 0)),
        ],
        out_specs=pl.BlockSpec((TBLK, H), lambda t, e: (t, 0)),
        out_shape=jax.ShapeDtypeStruct((T, H), jnp.float32),
    )(g_all, x_bf, up_all, down_all)
    return out


# speedup vs baseline: 1.6630x; 1.2192x over previous
import functools

import jax
import jax.numpy as jnp
from jax import lax
from jax.experimental import pallas as pl
from jax.experimental.pallas import tpu as pltpu
from jax.experimental.pallas import tpu_sc as plsc

H = 2048
E_DIM = 1024
N_SHARED = 2
N_ROUTED = 8
TOPK = 2
T = 4096
NP = T * TOPK

LANES = 128
TBLK = 256
VMAX = 64
NTILES = 32
TPT = T // NTILES
CH = 16


def _router_body(x_ref, w_ref, eidx_ref, prob_ref):
    logits = lax.dot_general(x_ref[...], w_ref[...], (((1,), (0,)), ((), ())),
                             preferred_element_type=jnp.float32)
    lane = lax.broadcasted_iota(jnp.int32, logits.shape, 1)
    valid = lane < N_ROUTED
    neg = jnp.full_like(logits, -jnp.inf)
    l = jnp.where(valid, logits, neg)
    m1 = jnp.max(l, axis=-1, keepdims=True)
    i1 = jnp.min(jnp.where(l == m1, lane, N_ROUTED + 7), axis=-1, keepdims=True)
    l2 = jnp.where(lane == i1, neg, l)
    m2 = jnp.max(l2, axis=-1, keepdims=True)
    i2 = jnp.min(jnp.where(l2 == m2, lane, N_ROUTED + 7), axis=-1, keepdims=True)
    z = jnp.sum(jnp.where(valid, jnp.exp(l - m1), 0.0), axis=-1, keepdims=True)
    p1 = 1.0 / z
    p2 = jnp.exp(m2 - m1) / z
    eidx_ref[...] = jnp.where(lane == 0, i1, jnp.where(lane == 1, i2, 0))
    prob_ref[...] = jnp.where(lane == 0, p1, jnp.where(lane == 1, p2, 0.0))


def _router(x, router_w):
    w_pad = jnp.zeros((H, LANES), jnp.float32).at[:, :N_ROUTED].set(router_w.T)
    eidx, prob = pl.pallas_call(
        _router_body,
        grid=(T // TBLK,),
        in_specs=[
            pl.BlockSpec((TBLK, H), lambda t: (t, 0)),
            pl.BlockSpec((H, LANES), lambda t: (0, 0)),
        ],
        out_specs=[
            pl.BlockSpec((TBLK, LANES), lambda t: (t, 0)),
            pl.BlockSpec((TBLK, LANES), lambda t: (t, 0)),
        ],
        out_shape=[
            jax.ShapeDtypeStruct((T, LANES), jnp.int32),
            jax.ShapeDtypeStruct((T, LANES), jnp.float32),
        ],
    )(x, w_pad)
    return eidx, prob


PPT2 = NP // 16


def _dispatch_body(ep_hbm, offs_hbm, pos_hbm,
                   ebuf, rankbuf, posbuf, cnt, offv):
    c = lax.axis_index("c")
    sid = lax.axis_index("s")
    iota16 = lax.iota(jnp.int32, 16)
    zeros16 = jnp.zeros((16,), jnp.int32)

    @pl.when(c == 0)
    def _work():
        base = sid * PPT2
        pltpu.sync_copy(ep_hbm, ebuf)
        cnt[...] = zeros16
        for sub in range(PPT2 // 16):
            ev = ebuf[pl.ds(base + sub * 16, 16)]
            cntv = cnt[...]
            basec = zeros16
            cadd = zeros16
            for e in range(N_ROUTED):
                m = ev == e
                cs = plsc.cumsum(jnp.where(m, 1, 0))
                basec = jnp.where(m, cntv[e] + cs - 1, basec)
                ce = jnp.max(cs)
                cadd = cadd + jnp.where(iota16 == e, ce, 0)
            rankbuf[pl.ds(sub * 16, 16)] = basec
            cnt[...] = cntv + cadd
        def chunk(i, carry):
            te, pre = carry
            ev = ebuf[pl.ds(i * 16, 16)]
            cvec = zeros16
            for e in range(N_ROUTED):
                pc = plsc.all_reduce_population_count(ev == e)
                cvec = cvec + jnp.where(iota16 == e, pc, 0)
            te = te + cvec
            pre = pre + jnp.where(i * 16 < base, cvec, zeros16)
            return te, pre

        te, pre = lax.fori_loop(0, NP // 16, chunk, (zeros16, zeros16))
        excl = plsc.cumsum(te) - te
        myoffv = excl + pre
        offv[...] = excl

        @pl.when(sid == 0)
        def _offs():
            pltpu.sync_copy(offv, offs_hbm)

        for sub in range(PPT2 // 16):
            ev = ebuf[pl.ds(base + sub * 16, 16)]
            basee = zeros16
            for e in range(N_ROUTED):
                basee = jnp.where(ev == e, myoffv[e], basee)
            posbuf[pl.ds(sub * 16, 16)] = basee + rankbuf[pl.ds(sub * 16, 16)]
        pltpu.sync_copy(posbuf, pos_hbm.at[pl.ds(base, PPT2)])


def _dispatch(ep):
    mesh = plsc.VectorSubcoreMesh(core_axis_name="c", subcore_axis_name="s")
    i32 = jnp.int32
    f = pl.kernel(
        _dispatch_body,
        out_type=[
            jax.ShapeDtypeStruct((16,), i32),
            jax.ShapeDtypeStruct((NP,), i32),
        ],
        mesh=mesh,
        scratch_types=[
            pltpu.VMEM((NP,), i32),
            pltpu.VMEM((PPT2,), i32),
            pltpu.VMEM((PPT2,), i32),
            pltpu.VMEM((16,), i32),
            pltpu.VMEM((16,), i32),
        ],
        compiler_params=pltpu.CompilerParams(needs_layout_passes=False),
    )
    return f(ep)


def _visits_body(offs_ref, vb_ref, ve_ref, vr0_ref, vr1_ref):
    def mk_expert(e, v):
        o0 = offs_ref[e]
        o1 = offs_ref[e + 1]
        b0 = o0 // TBLK
        bend = (o1 + TBLK - 1) // TBLK

        def body(b, v):
            vb_ref[v] = b
            ve_ref[v] = e
            vr0_ref[v] = jnp.maximum(o0 - b * TBLK, 0)
            vr1_ref[v] = jnp.minimum(o1 - b * TBLK, TBLK)
            return v + 1

        return lax.fori_loop(b0, bend, body, v)

    v = 0
    for e in range(N_ROUTED):
        v = mk_expert(e, v)
    last_b = vb_ref[v - 1]
    last_e = ve_ref[v - 1]

    def pad(i, _):
        vb_ref[i] = last_b
        ve_ref[i] = last_e
        vr0_ref[i] = 0
        vr1_ref[i] = 0
        return 0

    lax.fori_loop(v, VMAX, pad, 0)


def _visits(offs):
    i32 = jnp.int32
    smem = functools.partial(pl.BlockSpec, memory_space=pltpu.SMEM)
    return pl.pallas_call(
        _visits_body,
        in_specs=[smem()],
        out_specs=[smem(), smem(), smem(), smem()],
        out_shape=[jax.ShapeDtypeStruct((VMAX,), i32)] * 4,
    )(offs)


def _xscatter_body(x_hbm, pos_hbm, xg_hbm, xbuf, idx1, idx2, sem):
    c = lax.axis_index("c")
    sid = lax.axis_index("s")
    wid = sid * 2 + c
    for chv in range(TPT // CH):
        tb = wid * TPT + chv * CH
        pltpu.sync_copy(x_hbm.at[pl.ds(tb, CH)], xbuf)
        pltpu.sync_copy(pos_hbm.at[pl.ds(tb, CH)], idx1)
        pltpu.sync_copy(pos_hbm.at[pl.ds(T + tb, CH)], idx2)
        c1 = pltpu.async_copy(xbuf, xg_hbm.at[idx1], sem)
        c2 = pltpu.async_copy(xbuf, xg_hbm.at[idx2], sem)
        c1.wait()
        c2.wait()


def _xscatter(x, pos):
    mesh = plsc.VectorSubcoreMesh(core_axis_name="c", subcore_axis_name="s")
    f = pl.kernel(
        _xscatter_body,
        out_type=jax.ShapeDtypeStruct((NP, H), jnp.float32),
        mesh=mesh,
        scratch_types=[
            pltpu.VMEM((CH, H), jnp.float32),
            pltpu.VMEM((CH,), jnp.int32),
            pltpu.VMEM((CH,), jnp.int32),
            pltpu.SemaphoreType.DMA,
        ],
    )
    return f(x, pos)


def _gmm_body(vb_ref, ve_ref, vr0_ref, vr1_ref, xg_ref, up_ref, down_ref,
              ys_ref):
    v = pl.program_id(0)
    r0 = vr0_ref[v]
    r1 = vr1_ref[v]

    @pl.when(r1 > r0)
    def _():
        xb = xg_ref[...].astype(jnp.bfloat16)
        h = lax.dot_general(xb, up_ref[0], (((1,), (1,)), ((), ())),
                            preferred_element_type=jnp.float32)
        h = h * 0.5 * (1.0 + lax.erf(h * 0.7071067811865476))
        y = lax.dot_general(h.astype(jnp.bfloat16), down_ref[0],
                            (((1,), (1,)), ((), ())),
                            preferred_element_type=jnp.float32)
        rows = lax.broadcasted_iota(jnp.int32, (TBLK, H), 0)
        keep = (rows >= r0) & (rows < r1)
        ys_ref[...] = jnp.where(keep, y, ys_ref[...])


def _gmm(vb, ve, vr0, vr1, xg, up_bf, down_bf):
    grid_spec = pltpu.PrefetchScalarGridSpec(
        num_scalar_prefetch=4,
        grid=(VMAX,),
        in_specs=[
            pl.BlockSpec((TBLK, H), lambda v, vb, ve, r0, r1: (vb[v], 0)),
            pl.BlockSpec((1, E_DIM, H), lambda v, vb, ve, r0, r1: (ve[v], 0, 0)),
            pl.BlockSpec((1, H, E_DIM), lambda v, vb, ve, r0, r1: (ve[v], 0, 0)),
        ],
        out_specs=pl.BlockSpec((TBLK, H), lambda v, vb, ve, r0, r1: (vb[v], 0)),
    )
    return pl.pallas_call(
        _gmm_body,
        grid_spec=grid_spec,
        out_shape=jax.ShapeDtypeStruct((NP, H), jnp.float32),
    )(vb, ve, vr0, vr1, xg, up_bf, down_bf)


def _shared_body(x_ref, u_ref, d_ref, out_ref):
    h = lax.dot_general(x_ref[...], u_ref[...], (((1,), (1,)), ((), ())),
                        preferred_element_type=jnp.float32)
    h = h * 0.5 * (1.0 + lax.erf(h * 0.7071067811865476))
    out_ref[...] = lax.dot_general(h.astype(jnp.bfloat16), d_ref[...],
                                   (((1,), (1,)), ((), ())),
                                   preferred_element_type=jnp.float32)


def _shared(x_bf, u_bf, d_bf):
    su = N_SHARED * E_DIM
    return pl.pallas_call(
        _shared_body,
        grid=(T // TBLK,),
        in_specs=[
            pl.BlockSpec((TBLK, H), lambda t: (t, 0)),
            pl.BlockSpec((su, H), lambda t: (0, 0)),
            pl.BlockSpec((H, su), lambda t: (0, 0)),
        ],
        out_specs=pl.BlockSpec((TBLK, H), lambda t: (t, 0)),
        out_shape=jax.ShapeDtypeStruct((T, H), jnp.float32),
    )(x_bf, u_bf, d_bf)


def _combine_body(sh_hbm, ys_hbm, pos_hbm, wp_hbm, out_hbm,
                  sbuf, g1, g2, idx1, idx2, w1, w2, sem):
    c = lax.axis_index("c")
    sid = lax.axis_index("s")
    wid = sid * 2 + c
    for chv in range(TPT // CH):
        tb = wid * TPT + chv * CH
        pltpu.sync_copy(pos_hbm.at[pl.ds(tb, CH)], idx1)
        pltpu.sync_copy(pos_hbm.at[pl.ds(T + tb, CH)], idx2)
        cg1 = pltpu.async_copy(ys_hbm.at[idx1], g1, sem)
        cg2 = pltpu.async_copy(ys_hbm.at[idx2], g2, sem)
        pltpu.sync_copy(wp_hbm.at[pl.ds(tb, CH)], w1)
        pltpu.sync_copy(wp_hbm.at[pl.ds(T + tb, CH)], w2)
        pltpu.sync_copy(sh_hbm.at[pl.ds(tb, CH)], sbuf)
        cg1.wait()
        cg2.wait()
        wv1 = w1[...]
        wv2 = w2[...]
        for r in range(CH):
            wa = wv1[r]
            wb = wv2[r]

            def col(j, _):
                cs = pl.ds(j * 16, 16)
                sbuf[r, cs] = sbuf[r, cs] + wa * g1[r, cs] + wb * g2[r, cs]
                return 0

            lax.fori_loop(0, H // 16, col, 0)
        pltpu.sync_copy(sbuf, out_hbm.at[pl.ds(tb, CH)])


def _combine(sh, ys, pos, wp):
    mesh = plsc.VectorSubcoreMesh(core_axis_name="c", subcore_axis_name="s")
    f = pl.kernel(
        _combine_body,
        out_type=jax.ShapeDtypeStruct((T, H), jnp.float32),
        mesh=mesh,
        scratch_types=[
            pltpu.VMEM((CH, H), jnp.float32),
            pltpu.VMEM((CH, H), jnp.float32),
            pltpu.VMEM((CH, H), jnp.float32),
            pltpu.VMEM((CH,), jnp.int32),
            pltpu.VMEM((CH,), jnp.int32),
            pltpu.VMEM((CH,), jnp.float32),
            pltpu.VMEM((CH,), jnp.float32),
            pltpu.SemaphoreType.DMA,
        ],
    )
    return f(sh, ys, pos, wp)


def kernel(x, shared_up, shared_down, routed_up, routed_down, router_w):
    x_bf = x.astype(jnp.bfloat16)
    up_bf = routed_up.astype(jnp.bfloat16)
    down_bf = routed_down.astype(jnp.bfloat16)
    u_sh = shared_up.reshape(N_SHARED * E_DIM, H).astype(jnp.bfloat16)
    d_sh = jnp.concatenate([shared_down[i] for i in range(N_SHARED)],
                           axis=1).astype(jnp.bfloat16)

    eidx, prob = _router(x, router_w)
    ep = jnp.concatenate([eidx[:, 0], eidx[:, 1]])
    wp = jnp.concatenate([prob[:, 0], prob[:, 1]])

    offs, pos = _dispatch(ep)
    vb, ve, vr0, vr1 = _visits(offs)
    xg = _xscatter(x, pos)
    ys = _gmm(vb, ve, vr0, vr1, xg, up_bf, down_bf)
    sh = _shared(x_bf, u_sh, d_sh)
    out = _combine(sh, ys, pos, wp)
    return out


# speedup vs baseline: 1.7552x; 1.0554x over previous
import functools

import jax
import jax.numpy as jnp
from jax import lax
from jax.experimental import pallas as pl
from jax.experimental.pallas import tpu as pltpu
from jax.experimental.pallas import tpu_sc as plsc

H = 2048
E_DIM = 1024
N_SHARED = 2
N_ROUTED = 8
TOPK = 2
T = 4096
NP = T * TOPK

LANES = 128
TBLK = 256
VMAX = 64
NTILES = 32
TPT = T // NTILES
CH = 16


def _router_body(x_ref, w_ref, eidx_ref, prob_ref):
    logits = lax.dot_general(x_ref[...], w_ref[...], (((1,), (0,)), ((), ())),
                             preferred_element_type=jnp.float32)
    lane = lax.broadcasted_iota(jnp.int32, logits.shape, 1)
    valid = lane < N_ROUTED
    neg = jnp.full_like(logits, -jnp.inf)
    l = jnp.where(valid, logits, neg)
    m1 = jnp.max(l, axis=-1, keepdims=True)
    i1 = jnp.min(jnp.where(l == m1, lane, N_ROUTED + 7), axis=-1, keepdims=True)
    l2 = jnp.where(lane == i1, neg, l)
    m2 = jnp.max(l2, axis=-1, keepdims=True)
    i2 = jnp.min(jnp.where(l2 == m2, lane, N_ROUTED + 7), axis=-1, keepdims=True)
    z = jnp.sum(jnp.where(valid, jnp.exp(l - m1), 0.0), axis=-1, keepdims=True)
    p1 = 1.0 / z
    p2 = jnp.exp(m2 - m1) / z
    eidx_ref[...] = jnp.where(lane == 0, i1, jnp.where(lane == 1, i2, 0))
    prob_ref[...] = jnp.where(lane == 0, p1, jnp.where(lane == 1, p2, 0.0))


def _router(x, router_w):
    w_pad = jnp.zeros((H, LANES), jnp.float32).at[:, :N_ROUTED].set(router_w.T)
    eidx, prob = pl.pallas_call(
        _router_body,
        grid=(T // TBLK,),
        in_specs=[
            pl.BlockSpec((TBLK, H), lambda t: (t, 0)),
            pl.BlockSpec((H, LANES), lambda t: (0, 0)),
        ],
        out_specs=[
            pl.BlockSpec((TBLK, LANES), lambda t: (t, 0)),
            pl.BlockSpec((TBLK, LANES), lambda t: (t, 0)),
        ],
        out_shape=[
            jax.ShapeDtypeStruct((T, LANES), jnp.int32),
            jax.ShapeDtypeStruct((T, LANES), jnp.float32),
        ],
    )(x, w_pad)
    return eidx, prob


PPT2 = NP // 16


def _dispatch_body(ep_hbm, offs_hbm, pos_hbm,
                   ebuf, rankbuf, posbuf, cnt, offv):
    c = lax.axis_index("c")
    sid = lax.axis_index("s")
    iota16 = lax.iota(jnp.int32, 16)
    zeros16 = jnp.zeros((16,), jnp.int32)

    @pl.when(c == 0)
    def _work():
        base = sid * PPT2
        pltpu.sync_copy(ep_hbm, ebuf)
        cnt[...] = zeros16
        for sub in range(PPT2 // 16):
            ev = ebuf[pl.ds(base + sub * 16, 16)]
            cntv = cnt[...]
            basec = zeros16
            cadd = zeros16
            for e in range(N_ROUTED):
                m = ev == e
                cs = plsc.cumsum(jnp.where(m, 1, 0))
                basec = jnp.where(m, cntv[e] + cs - 1, basec)
                ce = jnp.max(cs)
                cadd = cadd + jnp.where(iota16 == e, ce, 0)
            rankbuf[pl.ds(sub * 16, 16)] = basec
            cnt[...] = cntv + cadd
        def chunk(i, carry):
            te, pre = carry
            ev = ebuf[pl.ds(i * 16, 16)]
            cvec = zeros16
            for e in range(N_ROUTED):
                pc = plsc.all_reduce_population_count(ev == e)
                cvec = cvec + jnp.where(iota16 == e, pc, 0)
            te = te + cvec
            pre = pre + jnp.where(i * 16 < base, cvec, zeros16)
            return te, pre

        te, pre = lax.fori_loop(0, NP // 16, chunk, (zeros16, zeros16))
        excl = plsc.cumsum(te) - te
        myoffv = excl + pre
        offv[...] = excl

        @pl.when(sid == 0)
        def _offs():
            pltpu.sync_copy(offv, offs_hbm)

        for sub in range(PPT2 // 16):
            ev = ebuf[pl.ds(base + sub * 16, 16)]
            basee = zeros16
            for e in range(N_ROUTED):
                basee = jnp.where(ev == e, myoffv[e], basee)
            posbuf[pl.ds(sub * 16, 16)] = basee + rankbuf[pl.ds(sub * 16, 16)]
        pltpu.sync_copy(posbuf, pos_hbm.at[pl.ds(base, PPT2)])


def _dispatch(ep):
    mesh = plsc.VectorSubcoreMesh(core_axis_name="c", subcore_axis_name="s")
    i32 = jnp.int32
    f = pl.kernel(
        _dispatch_body,
        out_type=[
            jax.ShapeDtypeStruct((16,), i32),
            jax.ShapeDtypeStruct((NP,), i32),
        ],
        mesh=mesh,
        scratch_types=[
            pltpu.VMEM((NP,), i32),
            pltpu.VMEM((PPT2,), i32),
            pltpu.VMEM((PPT2,), i32),
            pltpu.VMEM((16,), i32),
            pltpu.VMEM((16,), i32),
        ],
        compiler_params=pltpu.CompilerParams(needs_layout_passes=False),
    )
    return f(ep)


def _visits_body(offs_ref, vb_ref, ve_ref, vr0_ref, vr1_ref):
    def mk_expert(e, v):
        o0 = offs_ref[e]
        o1 = offs_ref[e + 1]
        b0 = o0 // TBLK
        bend = (o1 + TBLK - 1) // TBLK

        def body(b, v):
            vb_ref[v] = b
            ve_ref[v] = e
            vr0_ref[v] = jnp.maximum(o0 - b * TBLK, 0)
            vr1_ref[v] = jnp.minimum(o1 - b * TBLK, TBLK)
            return v + 1

        return lax.fori_loop(b0, bend, body, v)

    v = 0
    for e in range(N_ROUTED):
        v = mk_expert(e, v)
    last_b = vb_ref[v - 1]
    last_e = ve_ref[v - 1]

    def pad(i, _):
        vb_ref[i] = last_b
        ve_ref[i] = last_e
        vr0_ref[i] = 0
        vr1_ref[i] = 0
        return 0

    lax.fori_loop(v, VMAX, pad, 0)


def _visits(offs):
    i32 = jnp.int32
    smem = functools.partial(pl.BlockSpec, memory_space=pltpu.SMEM)
    return pl.pallas_call(
        _visits_body,
        in_specs=[smem()],
        out_specs=[smem(), smem(), smem(), smem()],
        out_shape=[jax.ShapeDtypeStruct((VMAX,), i32)] * 4,
    )(offs)


def _xscatter_body(x_hbm, pos_hbm, xg_hbm, xbuf, idx1, idx2,
                   sem_in0, sem_in1, sem_sc0, sem_sc1):
    c = lax.axis_index("c")
    sid = lax.axis_index("s")
    wid = sid * 2 + c
    nch = TPT // CH
    sem_in = (sem_in0, sem_in1)
    sem_sc = (sem_sc0, sem_sc1)

    def start_in(chv, s):
        tb = wid * TPT + chv * CH
        pltpu.async_copy(x_hbm.at[pl.ds(tb, CH)], xbuf.at[s], sem_in[s])
        pltpu.async_copy(pos_hbm.at[pl.ds(tb, CH)], idx1.at[s], sem_in[s])
        pltpu.async_copy(pos_hbm.at[pl.ds(T + tb, CH)], idx2.at[s], sem_in[s])

    def wait_in(chv, s):
        tb = wid * TPT + chv * CH
        pltpu.make_async_copy(x_hbm.at[pl.ds(tb, CH)], xbuf.at[s],
                              sem_in[s]).wait()
        pltpu.make_async_copy(pos_hbm.at[pl.ds(tb, CH)], idx1.at[s],
                              sem_in[s]).wait()
        pltpu.make_async_copy(pos_hbm.at[pl.ds(T + tb, CH)], idx2.at[s],
                              sem_in[s]).wait()

    def start_sc(s):
        pltpu.async_copy(xbuf.at[s], xg_hbm.at[idx1.at[s]], sem_sc[s])
        pltpu.async_copy(xbuf.at[s], xg_hbm.at[idx2.at[s]], sem_sc[s])

    def wait_sc(s):
        pltpu.make_async_copy(xbuf.at[s], xg_hbm.at[idx1.at[s]],
                              sem_sc[s]).wait()
        pltpu.make_async_copy(xbuf.at[s], xg_hbm.at[idx2.at[s]],
                              sem_sc[s]).wait()

    start_in(0, 0)
    for chv in range(nch):
        s = chv % 2
        wait_in(chv, s)
        start_sc(s)
        if chv + 1 < nch:
            if chv >= 1:
                wait_sc(1 - s)
            start_in(chv + 1, 1 - s)
    wait_sc((nch - 1) % 2)
    wait_sc(nch % 2)


def _xscatter(x, pos):
    mesh = plsc.VectorSubcoreMesh(core_axis_name="c", subcore_axis_name="s")
    f = pl.kernel(
        _xscatter_body,
        out_type=jax.ShapeDtypeStruct((NP, H), jnp.float32),
        mesh=mesh,
        scratch_types=[
            pltpu.VMEM((2, CH, H), jnp.float32),
            pltpu.VMEM((2, CH), jnp.int32),
            pltpu.VMEM((2, CH), jnp.int32),
            pltpu.SemaphoreType.DMA,
            pltpu.SemaphoreType.DMA,
            pltpu.SemaphoreType.DMA,
            pltpu.SemaphoreType.DMA,
        ],
        compiler_params=pltpu.CompilerParams(needs_layout_passes=False),
    )
    return f(x, pos)


def _gmm_body(vb_ref, ve_ref, vr0_ref, vr1_ref, xg_ref, up_ref, down_ref,
              ys_ref):
    v = pl.program_id(0)
    r0 = vr0_ref[v]
    r1 = vr1_ref[v]

    @pl.when(r1 > r0)
    def _():
        xb = xg_ref[...].astype(jnp.bfloat16)
        h = lax.dot_general(xb, up_ref[0], (((1,), (1,)), ((), ())),
                            preferred_element_type=jnp.float32)
        h = h * 0.5 * (1.0 + lax.erf(h * 0.7071067811865476))
        y = lax.dot_general(h.astype(jnp.bfloat16), down_ref[0],
                            (((1,), (1,)), ((), ())),
                            preferred_element_type=jnp.float32)
        rows = lax.broadcasted_iota(jnp.int32, (TBLK, H), 0)
        keep = (rows >= r0) & (rows < r1)
        ys_ref[...] = jnp.where(keep, y, ys_ref[...])


def _gmm(vb, ve, vr0, vr1, xg, up_bf, down_bf):
    grid_spec = pltpu.PrefetchScalarGridSpec(
        num_scalar_prefetch=4,
        grid=(VMAX,),
        in_specs=[
            pl.BlockSpec((TBLK, H), lambda v, vb, ve, r0, r1: (vb[v], 0)),
            pl.BlockSpec((1, E_DIM, H), lambda v, vb, ve, r0, r1: (ve[v], 0, 0)),
            pl.BlockSpec((1, H, E_DIM), lambda v, vb, ve, r0, r1: (ve[v], 0, 0)),
        ],
        out_specs=pl.BlockSpec((TBLK, H), lambda v, vb, ve, r0, r1: (vb[v], 0)),
    )
    return pl.pallas_call(
        _gmm_body,
        grid_spec=grid_spec,
        out_shape=jax.ShapeDtypeStruct((NP, H), jnp.float32),
    )(vb, ve, vr0, vr1, xg, up_bf, down_bf)


def _shared_body(x_ref, u_ref, d_ref, out_ref):
    h = lax.dot_general(x_ref[...], u_ref[...], (((1,), (1,)), ((), ())),
                        preferred_element_type=jnp.float32)
    h = h * 0.5 * (1.0 + lax.erf(h * 0.7071067811865476))
    out_ref[...] = lax.dot_general(h.astype(jnp.bfloat16), d_ref[...],
                                   (((1,), (1,)), ((), ())),
                                   preferred_element_type=jnp.float32)


def _shared(x_bf, u_bf, d_bf):
    su = N_SHARED * E_DIM
    return pl.pallas_call(
        _shared_body,
        grid=(T // TBLK,),
        in_specs=[
            pl.BlockSpec((TBLK, H), lambda t: (t, 0)),
            pl.BlockSpec((su, H), lambda t: (0, 0)),
            pl.BlockSpec((H, su), lambda t: (0, 0)),
        ],
        out_specs=pl.BlockSpec((TBLK, H), lambda t: (t, 0)),
        out_shape=jax.ShapeDtypeStruct((T, H), jnp.float32),
    )(x_bf, u_bf, d_bf)


CH5 = 8


def _combine_body(sh_hbm, ys_hbm, pos_hbm, wp_hbm, out_hbm,
                  sbuf, g1, g2, posb1, posb2, wb1, wb2,
                  sem_in0, sem_in1, sem_out0, sem_out1):
    c = lax.axis_index("c")
    sid = lax.axis_index("s")
    wid = sid * 2 + c
    tbase = wid * TPT
    nch = TPT // CH5
    sem_in = (sem_in0, sem_in1)
    sem_out = (sem_out0, sem_out1)

    pltpu.sync_copy(pos_hbm.at[pl.ds(tbase, TPT)], posb1)
    pltpu.sync_copy(pos_hbm.at[pl.ds(T + tbase, TPT)], posb2)
    pltpu.sync_copy(wp_hbm.at[pl.ds(tbase, TPT)], wb1.at[pl.ds(0, TPT)])
    pltpu.sync_copy(wp_hbm.at[pl.ds(T + tbase, TPT)], wb2.at[pl.ds(0, TPT)])

    def start_in(chv, s):
        tb = tbase + chv * CH5
        pltpu.async_copy(ys_hbm.at[posb1.at[pl.ds(chv * CH5, CH5)]],
                         g1.at[s], sem_in[s])
        pltpu.async_copy(ys_hbm.at[posb2.at[pl.ds(chv * CH5, CH5)]],
                         g2.at[s], sem_in[s])
        pltpu.async_copy(sh_hbm.at[pl.ds(tb, CH5)], sbuf.at[s], sem_in[s])

    def wait_in(chv, s):
        tb = tbase + chv * CH5
        pltpu.make_async_copy(ys_hbm.at[posb1.at[pl.ds(chv * CH5, CH5)]],
                              g1.at[s], sem_in[s]).wait()
        pltpu.make_async_copy(ys_hbm.at[posb2.at[pl.ds(chv * CH5, CH5)]],
                              g2.at[s], sem_in[s]).wait()
        pltpu.make_async_copy(sh_hbm.at[pl.ds(tb, CH5)], sbuf.at[s],
                              sem_in[s]).wait()

    def wait_out(chv, s):
        tb = tbase + chv * CH5
        pltpu.make_async_copy(sbuf.at[s], out_hbm.at[pl.ds(tb, CH5)],
                              sem_out[s]).wait()

    start_in(0, 0)
    for chv in range(nch):
        s = chv % 2
        if chv + 1 < nch:
            if chv >= 1:
                wait_out(chv - 1, 1 - s)
            start_in(chv + 1, 1 - s)
        wait_in(chv, s)
        wv1 = wb1[pl.ds(chv * CH5, 16)]
        wv2 = wb2[pl.ds(chv * CH5, 16)]
        iota16 = lax.iota(jnp.int32, 16)
        zf = jnp.zeros((16,), jnp.float32)

        def row(r, _):
            wa = jnp.sum(jnp.where(iota16 == r, wv1, zf))
            wb = jnp.sum(jnp.where(iota16 == r, wv2, zf))

            def col(j, _):
                for q in range(4):
                    cs = pl.ds(j * 64 + q * 16, 16)
                    sbuf.at[s][r, cs] = (sbuf.at[s][r, cs]
                                         + wa * g1.at[s][r, cs]
                                         + wb * g2.at[s][r, cs])
                return 0

            lax.fori_loop(0, H // 64, col, 0)
            return 0

        lax.fori_loop(0, CH5, row, 0)
        tb = tbase + chv * CH5
        pltpu.async_copy(sbuf.at[s], out_hbm.at[pl.ds(tb, CH5)], sem_out[s])
    wait_out(nch - 2, nch % 2)
    wait_out(nch - 1, (nch - 1) % 2)


def _combine(sh, ys, pos, wp):
    mesh = plsc.VectorSubcoreMesh(core_axis_name="c", subcore_axis_name="s")
    f32 = jnp.float32
    f = pl.kernel(
        _combine_body,
        out_type=jax.ShapeDtypeStruct((T, H), f32),
        mesh=mesh,
        scratch_types=[
            pltpu.VMEM((2, CH5, H), f32),
            pltpu.VMEM((2, CH5, H), f32),
            pltpu.VMEM((2, CH5, H), f32),
            pltpu.VMEM((TPT,), jnp.int32),
            pltpu.VMEM((TPT,), jnp.int32),
            pltpu.VMEM((TPT + 16,), f32),
            pltpu.VMEM((TPT + 16,), f32),
            pltpu.SemaphoreType.DMA,
            pltpu.SemaphoreType.DMA,
            pltpu.SemaphoreType.DMA,
            pltpu.SemaphoreType.DMA,
        ],
        compiler_params=pltpu.CompilerParams(needs_layout_passes=False),
    )
    return f(sh, ys, pos, wp)


def kernel(x, shared_up, shared_down, routed_up, routed_down, router_w):
    x_bf = x.astype(jnp.bfloat16)
    up_bf = routed_up.astype(jnp.bfloat16)
    down_bf = routed_down.astype(jnp.bfloat16)
    u_sh = shared_up.reshape(N_SHARED * E_DIM, H).astype(jnp.bfloat16)
    d_sh = jnp.concatenate([shared_down[i] for i in range(N_SHARED)],
                           axis=1).astype(jnp.bfloat16)

    eidx, prob = _router(x, router_w)
    ep = jnp.concatenate([eidx[:, 0], eidx[:, 1]])
    wp = jnp.concatenate([prob[:, 0], prob[:, 1]])

    offs, pos = _dispatch(ep)
    vb, ve, vr0, vr1 = _visits(offs)
    xg = _xscatter(x, pos)
    ys = _gmm(vb, ve, vr0, vr1, xg, up_bf, down_bf)
    sh = _shared(x_bf, u_sh, d_sh)
    out = _combine(sh, ys, pos, wp)
    return out


# speedup vs baseline: 1.8894x; 1.0765x over previous
import functools

import jax
import jax.numpy as jnp
from jax import lax
from jax.experimental import pallas as pl
from jax.experimental.pallas import tpu as pltpu
from jax.experimental.pallas import tpu_sc as plsc

H = 2048
E_DIM = 1024
N_SHARED = 2
N_ROUTED = 8
TOPK = 2
T = 4096
NP = T * TOPK

LANES = 128
TBLK = 256
VMAX = 48
NTILES = 32
TPT = T // NTILES
CH = 16


def _router_body(x_ref, w_ref, eidx_ref, prob_ref):
    logits = lax.dot_general(x_ref[...], w_ref[...], (((1,), (0,)), ((), ())),
                             preferred_element_type=jnp.float32)
    lane = lax.broadcasted_iota(jnp.int32, logits.shape, 1)
    valid = lane < N_ROUTED
    neg = jnp.full_like(logits, -jnp.inf)
    l = jnp.where(valid, logits, neg)
    m1 = jnp.max(l, axis=-1, keepdims=True)
    i1 = jnp.min(jnp.where(l == m1, lane, N_ROUTED + 7), axis=-1, keepdims=True)
    l2 = jnp.where(lane == i1, neg, l)
    m2 = jnp.max(l2, axis=-1, keepdims=True)
    i2 = jnp.min(jnp.where(l2 == m2, lane, N_ROUTED + 7), axis=-1, keepdims=True)
    z = jnp.sum(jnp.where(valid, jnp.exp(l - m1), 0.0), axis=-1, keepdims=True)
    p1 = 1.0 / z
    p2 = jnp.exp(m2 - m1) / z
    eidx_ref[...] = jnp.where(lane == 0, i1, jnp.where(lane == 1, i2, 0))
    prob_ref[...] = jnp.where(lane == 0, p1, jnp.where(lane == 1, p2, 0.0))


def _router(x, router_w):
    w_pad = jnp.zeros((H, LANES), jnp.float32).at[:, :N_ROUTED].set(router_w.T)
    eidx, prob = pl.pallas_call(
        _router_body,
        grid=(T // TBLK,),
        in_specs=[
            pl.BlockSpec((TBLK, H), lambda t: (t, 0)),
            pl.BlockSpec((H, LANES), lambda t: (0, 0)),
        ],
        out_specs=[
            pl.BlockSpec((TBLK, LANES), lambda t: (t, 0)),
            pl.BlockSpec((TBLK, LANES), lambda t: (t, 0)),
        ],
        out_shape=[
            jax.ShapeDtypeStruct((T, LANES), jnp.int32),
            jax.ShapeDtypeStruct((T, LANES), jnp.float32),
        ],
    )(x, w_pad)
    return eidx, prob


PPT2 = NP // 16


def _dispatch_body(ep_hbm, pos_hbm, vb_hbm, ve_hbm, vr0_hbm, vr1_hbm,
                   ebuf, rankbuf, posbuf, cnt, vbv, vev, vr0v, vr1v):
    c = lax.axis_index("c")
    sid = lax.axis_index("s")
    iota16 = lax.iota(jnp.int32, 16)
    zeros16 = jnp.zeros((16,), jnp.int32)

    @pl.when(c == 0)
    def _work():
        base = sid * PPT2
        pltpu.sync_copy(ep_hbm, ebuf)
        cnt[...] = zeros16
        for sub in range(PPT2 // 16):
            ev = ebuf[pl.ds(base + sub * 16, 16)]
            cntv = cnt[...]
            basec = zeros16
            cadd = zeros16
            for e in range(N_ROUTED):
                m = ev == e
                cs = plsc.cumsum(jnp.where(m, 1, 0))
                basec = jnp.where(m, cntv[e] + cs - 1, basec)
                ce = jnp.max(cs)
                cadd = cadd + jnp.where(iota16 == e, ce, 0)
            rankbuf[pl.ds(sub * 16, 16)] = basec
            cnt[...] = cntv + cadd
        def chunk(i, carry):
            te, pre = carry
            ev = ebuf[pl.ds(i * 16, 16)]
            cvec = zeros16
            for e in range(N_ROUTED):
                pc = plsc.all_reduce_population_count(ev == e)
                cvec = cvec + jnp.where(iota16 == e, pc, 0)
            te = te + cvec
            pre = pre + jnp.where(i * 16 < base, cvec, zeros16)
            return te, pre

        te, pre = lax.fori_loop(0, NP // 16, chunk, (zeros16, zeros16))
        excl = plsc.cumsum(te) - te
        myoffv = excl + pre

        @pl.when(sid == 0)
        def _visit_tables():
            o = [excl[e] for e in range(N_ROUTED)] + [jnp.int32(NP)]
            b0 = [o[e] // TBLK for e in range(N_ROUTED)]
            bend = [(o[e + 1] + TBLK - 1) // TBLK for e in range(N_ROUTED)]
            nb = [bend[e] - b0[e] for e in range(N_ROUTED)]
            V = [jnp.int32(0)]
            for e in range(N_ROUTED):
                V.append(V[-1] + nb[e])
            vtot = V[N_ROUTED]
            last_b = jnp.int32(0)
            last_e = jnp.int32(0)
            for e in range(N_ROUTED):
                nz = nb[e] > 0
                last_b = jnp.where(nz, bend[e] - 1, last_b)
                last_e = jnp.where(nz, e, last_e)
            for g in range(VMAX // 16):
                v = iota16 + 16 * g
                b = zeros16
                el = zeros16
                r0v = zeros16
                r1v = zeros16
                for e in range(N_ROUTED):
                    m = (v >= V[e]) & (v < V[e] + nb[e])
                    bb = b0[e] + (v - V[e])
                    rr0 = jnp.maximum(o[e] - bb * TBLK, 0)
                    rr1 = jnp.minimum(o[e + 1] - bb * TBLK, TBLK)
                    b = jnp.where(m, bb, b)
                    el = jnp.where(m, e, el)
                    r0v = jnp.where(m, rr0, r0v)
                    r1v = jnp.where(m, rr1, r1v)
                pad = v >= vtot
                b = jnp.where(pad, last_b, b)
                el = jnp.where(pad, last_e, el)
                r0v = jnp.where(pad, 0, r0v)
                r1v = jnp.where(pad, 0, r1v)
                sl = pl.ds(g * 16, 16)
                vbv[sl] = b
                vev[sl] = el
                vr0v[sl] = r0v
                vr1v[sl] = r1v
            pltpu.sync_copy(vbv, vb_hbm)
            pltpu.sync_copy(vev, ve_hbm)
            pltpu.sync_copy(vr0v, vr0_hbm)
            pltpu.sync_copy(vr1v, vr1_hbm)

        for sub in range(PPT2 // 16):
            ev = ebuf[pl.ds(base + sub * 16, 16)]
            basee = zeros16
            for e in range(N_ROUTED):
                basee = jnp.where(ev == e, myoffv[e], basee)
            posbuf[pl.ds(sub * 16, 16)] = basee + rankbuf[pl.ds(sub * 16, 16)]
        pltpu.sync_copy(posbuf, pos_hbm.at[pl.ds(base, PPT2)])


def _dispatch(ep):
    mesh = plsc.VectorSubcoreMesh(core_axis_name="c", subcore_axis_name="s")
    i32 = jnp.int32
    f = pl.kernel(
        _dispatch_body,
        out_type=[
            jax.ShapeDtypeStruct((NP,), i32),
            jax.ShapeDtypeStruct((VMAX,), i32),
            jax.ShapeDtypeStruct((VMAX,), i32),
            jax.ShapeDtypeStruct((VMAX,), i32),
            jax.ShapeDtypeStruct((VMAX,), i32),
        ],
        mesh=mesh,
        scratch_types=[
            pltpu.VMEM((NP,), i32),
            pltpu.VMEM((PPT2,), i32),
            pltpu.VMEM((PPT2,), i32),
            pltpu.VMEM((16,), i32),
            pltpu.VMEM((VMAX,), i32),
            pltpu.VMEM((VMAX,), i32),
            pltpu.VMEM((VMAX,), i32),
            pltpu.VMEM((VMAX,), i32),
        ],
        compiler_params=pltpu.CompilerParams(needs_layout_passes=False),
    )
    return f(ep)


def _visits_body(offs_ref, vb_ref, ve_ref, vr0_ref, vr1_ref):
    def mk_expert(e, v):
        o0 = offs_ref[e]
        o1 = offs_ref[e + 1]
        b0 = o0 // TBLK
        bend = (o1 + TBLK - 1) // TBLK

        def body(b, v):
            vb_ref[v] = b
            ve_ref[v] = e
            vr0_ref[v] = jnp.maximum(o0 - b * TBLK, 0)
            vr1_ref[v] = jnp.minimum(o1 - b * TBLK, TBLK)
            return v + 1

        return lax.fori_loop(b0, bend, body, v)

    v = 0
    for e in range(N_ROUTED):
        v = mk_expert(e, v)
    last_b = vb_ref[v - 1]
    last_e = ve_ref[v - 1]

    def pad(i, _):
        vb_ref[i] = last_b
        ve_ref[i] = last_e
        vr0_ref[i] = 0
        vr1_ref[i] = 0
        return 0

    lax.fori_loop(v, VMAX, pad, 0)


def _visits(offs):
    i32 = jnp.int32
    smem = functools.partial(pl.BlockSpec, memory_space=pltpu.SMEM)
    return pl.pallas_call(
        _visits_body,
        in_specs=[smem()],
        out_specs=[smem(), smem(), smem(), smem()],
        out_shape=[jax.ShapeDtypeStruct((VMAX,), i32)] * 4,
    )(offs)


def _xscatter_body(x_hbm, pos_hbm, xg_hbm, xbuf, idx1, idx2,
                   sem_in0, sem_in1, sem_sc0, sem_sc1):
    c = lax.axis_index("c")
    sid = lax.axis_index("s")
    wid = sid * 2 + c
    nch = TPT // CH
    sem_in = (sem_in0, sem_in1)
    sem_sc = (sem_sc0, sem_sc1)

    def start_in(chv, s):
        tb = wid * TPT + chv * CH
        pltpu.async_copy(x_hbm.at[pl.ds(tb, CH)], xbuf.at[s], sem_in[s])
        pltpu.async_copy(pos_hbm.at[pl.ds(tb, CH)], idx1.at[s], sem_in[s])
        pltpu.async_copy(pos_hbm.at[pl.ds(T + tb, CH)], idx2.at[s], sem_in[s])

    def wait_in(chv, s):
        tb = wid * TPT + chv * CH
        pltpu.make_async_copy(x_hbm.at[pl.ds(tb, CH)], xbuf.at[s],
                              sem_in[s]).wait()
        pltpu.make_async_copy(pos_hbm.at[pl.ds(tb, CH)], idx1.at[s],
                              sem_in[s]).wait()
        pltpu.make_async_copy(pos_hbm.at[pl.ds(T + tb, CH)], idx2.at[s],
                              sem_in[s]).wait()

    def start_sc(s):
        pltpu.async_copy(xbuf.at[s], xg_hbm.at[idx1.at[s]], sem_sc[s])
        pltpu.async_copy(xbuf.at[s], xg_hbm.at[idx2.at[s]], sem_sc[s])

    def wait_sc(s):
        pltpu.make_async_copy(xbuf.at[s], xg_hbm.at[idx1.at[s]],
                              sem_sc[s]).wait()
        pltpu.make_async_copy(xbuf.at[s], xg_hbm.at[idx2.at[s]],
                              sem_sc[s]).wait()

    start_in(0, 0)
    for chv in range(nch):
        s = chv % 2
        wait_in(chv, s)
        start_sc(s)
        if chv + 1 < nch:
            if chv >= 1:
                wait_sc(1 - s)
            start_in(chv + 1, 1 - s)
    wait_sc((nch - 1) % 2)
    wait_sc(nch % 2)


def _xscatter(x, pos):
    mesh = plsc.VectorSubcoreMesh(core_axis_name="c", subcore_axis_name="s")
    f = pl.kernel(
        _xscatter_body,
        out_type=jax.ShapeDtypeStruct((NP, H), jnp.float32),
        mesh=mesh,
        scratch_types=[
            pltpu.VMEM((2, CH, H), jnp.float32),
            pltpu.VMEM((2, CH), jnp.int32),
            pltpu.VMEM((2, CH), jnp.int32),
            pltpu.SemaphoreType.DMA,
            pltpu.SemaphoreType.DMA,
            pltpu.SemaphoreType.DMA,
            pltpu.SemaphoreType.DMA,
        ],
        compiler_params=pltpu.CompilerParams(needs_layout_passes=False),
    )
    return f(x, pos)


def _gmm_body(vb_ref, ve_ref, vr0_ref, vr1_ref, xg_ref, up_ref, down_ref,
              ys_ref):
    v = pl.program_id(0)
    r0 = vr0_ref[v]
    r1 = vr1_ref[v]

    @pl.when(r1 > r0)
    def _():
        xb = xg_ref[...].astype(jnp.bfloat16)
        h = lax.dot_general(xb, up_ref[0], (((1,), (1,)), ((), ())),
                            preferred_element_type=jnp.float32)
        h = h * 0.5 * (1.0 + lax.erf(h * 0.7071067811865476))
        y = lax.dot_general(h.astype(jnp.bfloat16), down_ref[0],
                            (((1,), (1,)), ((), ())),
                            preferred_element_type=jnp.float32)
        rows = lax.broadcasted_iota(jnp.int32, (TBLK, H), 0)
        keep = (rows >= r0) & (rows < r1)
        ys_ref[...] = jnp.where(keep, y, ys_ref[...])


def _gmm(vb, ve, vr0, vr1, xg, up_bf, down_bf):
    grid_spec = pltpu.PrefetchScalarGridSpec(
        num_scalar_prefetch=4,
        grid=(VMAX,),
        in_specs=[
            pl.BlockSpec((TBLK, H), lambda v, vb, ve, r0, r1: (vb[v], 0)),
            pl.BlockSpec((1, E_DIM, H), lambda v, vb, ve, r0, r1: (ve[v], 0, 0)),
            pl.BlockSpec((1, H, E_DIM), lambda v, vb, ve, r0, r1: (ve[v], 0, 0)),
        ],
        out_specs=pl.BlockSpec((TBLK, H), lambda v, vb, ve, r0, r1: (vb[v], 0)),
    )
    return pl.pallas_call(
        _gmm_body,
        grid_spec=grid_spec,
        out_shape=jax.ShapeDtypeStruct((NP, H), jnp.float32),
    )(vb, ve, vr0, vr1, xg, up_bf, down_bf)


def _shared_body(x_ref, u_ref, d_ref, w_ref, out_ref, eidx_ref, prob_ref):
    _router_body(x_ref, w_ref, eidx_ref, prob_ref)
    xb = x_ref[...].astype(jnp.bfloat16)
    h = lax.dot_general(xb, u_ref[...], (((1,), (1,)), ((), ())),
                        preferred_element_type=jnp.float32)
    h = h * 0.5 * (1.0 + lax.erf(h * 0.7071067811865476))
    out_ref[...] = lax.dot_general(h.astype(jnp.bfloat16), d_ref[...],
                                   (((1,), (1,)), ((), ())),
                                   preferred_element_type=jnp.float32)


def _shared_router(x, u_bf, d_bf, router_w):
    su = N_SHARED * E_DIM
    w_pad = jnp.zeros((H, LANES), jnp.float32).at[:, :N_ROUTED].set(router_w.T)
    return pl.pallas_call(
        _shared_body,
        grid=(T // TBLK,),
        in_specs=[
            pl.BlockSpec((TBLK, H), lambda t: (t, 0)),
            pl.BlockSpec((su, H), lambda t: (0, 0)),
            pl.BlockSpec((H, su), lambda t: (0, 0)),
            pl.BlockSpec((H, LANES), lambda t: (0, 0)),
        ],
        out_specs=[
            pl.BlockSpec((TBLK, H), lambda t: (t, 0)),
            pl.BlockSpec((TBLK, LANES), lambda t: (t, 0)),
            pl.BlockSpec((TBLK, LANES), lambda t: (t, 0)),
        ],
        out_shape=[
            jax.ShapeDtypeStruct((T, H), jnp.float32),
            jax.ShapeDtypeStruct((T, LANES), jnp.int32),
            jax.ShapeDtypeStruct((T, LANES), jnp.float32),
        ],
    )(x, u_bf, d_bf, w_pad)


CH5 = 8


def _combine_body(sh_hbm, ys_hbm, pos_hbm, wp_hbm, out_hbm,
                  sbuf, g1, g2, posb1, posb2, wb1, wb2,
                  sem_in0, sem_in1, sem_out0, sem_out1):
    c = lax.axis_index("c")
    sid = lax.axis_index("s")
    wid = sid * 2 + c
    tbase = wid * TPT
    nch = TPT // CH5
    sem_in = (sem_in0, sem_in1)
    sem_out = (sem_out0, sem_out1)

    pltpu.sync_copy(pos_hbm.at[pl.ds(tbase, TPT)], posb1)
    pltpu.sync_copy(pos_hbm.at[pl.ds(T + tbase, TPT)], posb2)
    pltpu.sync_copy(wp_hbm.at[pl.ds(tbase, TPT)], wb1.at[pl.ds(0, TPT)])
    pltpu.sync_copy(wp_hbm.at[pl.ds(T + tbase, TPT)], wb2.at[pl.ds(0, TPT)])

    def start_in(chv, s):
        tb = tbase + chv * CH5
        pltpu.async_copy(ys_hbm.at[posb1.at[pl.ds(chv * CH5, CH5)]],
                         g1.at[s], sem_in[s])
        pltpu.async_copy(ys_hbm.at[posb2.at[pl.ds(chv * CH5, CH5)]],
                         g2.at[s], sem_in[s])
        pltpu.async_copy(sh_hbm.at[pl.ds(tb, CH5)], sbuf.at[s], sem_in[s])

    def wait_in(chv, s):
        tb = tbase + chv * CH5
        pltpu.make_async_copy(ys_hbm.at[posb1.at[pl.ds(chv * CH5, CH5)]],
                              g1.at[s], sem_in[s]).wait()
        pltpu.make_async_copy(ys_hbm.at[posb2.at[pl.ds(chv * CH5, CH5)]],
                              g2.at[s], sem_in[s]).wait()
        pltpu.make_async_copy(sh_hbm.at[pl.ds(tb, CH5)], sbuf.at[s],
                              sem_in[s]).wait()

    def wait_out(chv, s):
        tb = tbase + chv * CH5
        pltpu.make_async_copy(sbuf.at[s], out_hbm.at[pl.ds(tb, CH5)],
                              sem_out[s]).wait()

    start_in(0, 0)
    for chv in range(nch):
        s = chv % 2
        if chv + 1 < nch:
            if chv >= 1:
                wait_out(chv - 1, 1 - s)
            start_in(chv + 1, 1 - s)
        wait_in(chv, s)
        wv1 = wb1[pl.ds(chv * CH5, 16)]
        wv2 = wb2[pl.ds(chv * CH5, 16)]
        iota16 = lax.iota(jnp.int32, 16)
        zf = jnp.zeros((16,), jnp.float32)

        def row(r, _):
            wa = jnp.sum(jnp.where(iota16 == r, wv1, zf))
            wb = jnp.sum(jnp.where(iota16 == r, wv2, zf))

            def col(j, _):
                for q in range(4):
                    cs = pl.ds(j * 64 + q * 16, 16)
                    sbuf.at[s][r, cs] = (sbuf.at[s][r, cs]
                                         + wa * g1.at[s][r, cs]
                                         + wb * g2.at[s][r, cs])
                return 0

            lax.fori_loop(0, H // 64, col, 0)
            return 0

        lax.fori_loop(0, CH5, row, 0)
        tb = tbase + chv * CH5
        pltpu.async_copy(sbuf.at[s], out_hbm.at[pl.ds(tb, CH5)], sem_out[s])
    wait_out(nch - 2, nch % 2)
    wait_out(nch - 1, (nch - 1) % 2)


def _combine(sh, ys, pos, wp):
    mesh = plsc.VectorSubcoreMesh(core_axis_name="c", subcore_axis_name="s")
    f32 = jnp.float32
    f = pl.kernel(
        _combine_body,
        out_type=jax.ShapeDtypeStruct((T, H), f32),
        mesh=mesh,
        scratch_types=[
            pltpu.VMEM((2, CH5, H), f32),
            pltpu.VMEM((2, CH5, H), f32),
            pltpu.VMEM((2, CH5, H), f32),
            pltpu.VMEM((TPT,), jnp.int32),
            pltpu.VMEM((TPT,), jnp.int32),
            pltpu.VMEM((TPT + 16,), f32),
            pltpu.VMEM((TPT + 16,), f32),
            pltpu.SemaphoreType.DMA,
            pltpu.SemaphoreType.DMA,
            pltpu.SemaphoreType.DMA,
            pltpu.SemaphoreType.DMA,
        ],
        compiler_params=pltpu.CompilerParams(needs_layout_passes=False),
    )
    return f(sh, ys, pos, wp)


def kernel(x, shared_up, shared_down, routed_up, routed_down, router_w):
    up_bf = routed_up.astype(jnp.bfloat16)
    down_bf = routed_down.astype(jnp.bfloat16)
    u_sh = shared_up.reshape(N_SHARED * E_DIM, H).astype(jnp.bfloat16)
    d_sh = jnp.concatenate([shared_down[i] for i in range(N_SHARED)],
                           axis=1).astype(jnp.bfloat16)

    sh, eidx, prob = _shared_router(x, u_sh, d_sh, router_w)
    ep = jnp.concatenate([eidx[:, 0], eidx[:, 1]])
    wp = jnp.concatenate([prob[:, 0], prob[:, 1]])

    pos, vb, ve, vr0, vr1 = _dispatch(ep)
    xg = _xscatter(x, pos)
    ys = _gmm(vb, ve, vr0, vr1, xg, up_bf, down_bf)
    out = _combine(sh, ys, pos, wp)
    return out


# speedup vs baseline: 1.8981x; 1.0046x over previous
import functools

import jax
import jax.numpy as jnp
from jax import lax
from jax.experimental import pallas as pl
from jax.experimental.pallas import tpu as pltpu
from jax.experimental.pallas import tpu_sc as plsc

H = 2048
E_DIM = 1024
N_SHARED = 2
N_ROUTED = 8
TOPK = 2
T = 4096
NP = T * TOPK

LANES = 128
TBLK = 256
VMAX = 48
NTILES = 32
TPT = T // NTILES
CH = 16


def _router_body(x_ref, w_ref, eidx_ref, prob_ref):
    logits = lax.dot_general(x_ref[...], w_ref[...], (((1,), (0,)), ((), ())),
                             preferred_element_type=jnp.float32)
    lane = lax.broadcasted_iota(jnp.int32, logits.shape, 1)
    valid = lane < N_ROUTED
    neg = jnp.full_like(logits, -jnp.inf)
    l = jnp.where(valid, logits, neg)
    m1 = jnp.max(l, axis=-1, keepdims=True)
    i1 = jnp.min(jnp.where(l == m1, lane, N_ROUTED + 7), axis=-1, keepdims=True)
    l2 = jnp.where(lane == i1, neg, l)
    m2 = jnp.max(l2, axis=-1, keepdims=True)
    i2 = jnp.min(jnp.where(l2 == m2, lane, N_ROUTED + 7), axis=-1, keepdims=True)
    z = jnp.sum(jnp.where(valid, jnp.exp(l - m1), 0.0), axis=-1, keepdims=True)
    p1 = 1.0 / z
    p2 = jnp.exp(m2 - m1) / z
    eidx_ref[...] = jnp.where(lane == 0, i1, jnp.where(lane == 1, i2, 0))
    prob_ref[...] = jnp.where(lane == 0, p1, jnp.where(lane == 1, p2, 0.0))


def _router(x, router_w):
    w_pad = jnp.zeros((H, LANES), jnp.float32).at[:, :N_ROUTED].set(router_w.T)
    eidx, prob = pl.pallas_call(
        _router_body,
        grid=(T // TBLK,),
        in_specs=[
            pl.BlockSpec((TBLK, H), lambda t: (t, 0)),
            pl.BlockSpec((H, LANES), lambda t: (0, 0)),
        ],
        out_specs=[
            pl.BlockSpec((TBLK, LANES), lambda t: (t, 0)),
            pl.BlockSpec((TBLK, LANES), lambda t: (t, 0)),
        ],
        out_shape=[
            jax.ShapeDtypeStruct((T, LANES), jnp.int32),
            jax.ShapeDtypeStruct((T, LANES), jnp.float32),
        ],
    )(x, w_pad)
    return eidx, prob


PPT2 = NP // NTILES


def _dispatch_body(ep_hbm, x_hbm, pos_hbm, vb_hbm, ve_hbm, vr0_hbm, vr1_hbm,
                   xg_hbm, ebuf, rankbuf, posbuf, cnt, vbv, vev, vr0v, vr1v,
                   xbuf, idxb, sem_in0, sem_in1, sem_sc0, sem_sc1):
    c = lax.axis_index("c")
    sid = lax.axis_index("s")
    wid = sid * 2 + c
    iota16 = lax.iota(jnp.int32, 16)
    zeros16 = jnp.zeros((16,), jnp.int32)

    if True:
        base = wid * PPT2
        pltpu.sync_copy(ep_hbm, ebuf)
        cnt[...] = zeros16
        for sub in range(PPT2 // 16):
            ev = ebuf[pl.ds(base + sub * 16, 16)]
            cntv = cnt[...]
            basec = zeros16
            cadd = zeros16
            for e in range(N_ROUTED):
                m = ev == e
                cs = plsc.cumsum(jnp.where(m, 1, 0))
                basec = jnp.where(m, cntv[e] + cs - 1, basec)
                ce = jnp.max(cs)
                cadd = cadd + jnp.where(iota16 == e, ce, 0)
            rankbuf[pl.ds(sub * 16, 16)] = basec
            cnt[...] = cntv + cadd
        def chunk(i, carry):
            te, pre = carry
            ev = ebuf[pl.ds(i * 16, 16)]
            cvec = zeros16
            for e in range(N_ROUTED):
                pc = plsc.all_reduce_population_count(ev == e)
                cvec = cvec + jnp.where(iota16 == e, pc, 0)
            te = te + cvec
            pre = pre + jnp.where(i * 16 < base, cvec, zeros16)
            return te, pre

        te, pre = lax.fori_loop(0, NP // 16, chunk, (zeros16, zeros16))
        excl = plsc.cumsum(te) - te
        myoffv = excl + pre

        @pl.when(wid == 0)
        def _visit_tables():
            o = [excl[e] for e in range(N_ROUTED)] + [jnp.int32(NP)]
            b0 = [o[e] // TBLK for e in range(N_ROUTED)]
            bend = [(o[e + 1] + TBLK - 1) // TBLK for e in range(N_ROUTED)]
            nb = [bend[e] - b0[e] for e in range(N_ROUTED)]
            V = [jnp.int32(0)]
            for e in range(N_ROUTED):
                V.append(V[-1] + nb[e])
            vtot = V[N_ROUTED]
            last_b = jnp.int32(0)
            last_e = jnp.int32(0)
            for e in range(N_ROUTED):
                nz = nb[e] > 0
                last_b = jnp.where(nz, bend[e] - 1, last_b)
                last_e = jnp.where(nz, e, last_e)
            for g in range(VMAX // 16):
                v = iota16 + 16 * g
                b = zeros16
                el = zeros16
                r0v = zeros16
                r1v = zeros16
                for e in range(N_ROUTED):
                    m = (v >= V[e]) & (v < V[e] + nb[e])
                    bb = b0[e] + (v - V[e])
                    rr0 = jnp.maximum(o[e] - bb * TBLK, 0)
                    rr1 = jnp.minimum(o[e + 1] - bb * TBLK, TBLK)
                    b = jnp.where(m, bb, b)
                    el = jnp.where(m, e, el)
                    r0v = jnp.where(m, rr0, r0v)
                    r1v = jnp.where(m, rr1, r1v)
                pad = v >= vtot
                b = jnp.where(pad, last_b, b)
                el = jnp.where(pad, last_e, el)
                r0v = jnp.where(pad, 0, r0v)
                r1v = jnp.where(pad, 0, r1v)
                sl = pl.ds(g * 16, 16)
                vbv[sl] = b
                vev[sl] = el
                vr0v[sl] = r0v
                vr1v[sl] = r1v
            pltpu.sync_copy(vbv, vb_hbm)
            pltpu.sync_copy(vev, ve_hbm)
            pltpu.sync_copy(vr0v, vr0_hbm)
            pltpu.sync_copy(vr1v, vr1_hbm)

        for sub in range(PPT2 // 16):
            ev = ebuf[pl.ds(base + sub * 16, 16)]
            basee = zeros16
            for e in range(N_ROUTED):
                basee = jnp.where(ev == e, myoffv[e], basee)
            posbuf[pl.ds(sub * 16, 16)] = basee + rankbuf[pl.ds(sub * 16, 16)]
        pltpu.sync_copy(posbuf, pos_hbm.at[pl.ds(base, PPT2)])

        tokb = base - jnp.where(base >= T, T, 0)
        nch = PPT2 // CH
        sem_in = (sem_in0, sem_in1)
        sem_sc = (sem_sc0, sem_sc1)

        def start_in(chv, s):
            pltpu.async_copy(x_hbm.at[pl.ds(tokb + chv * CH, CH)],
                             xbuf.at[s], sem_in[s])

        def wait_in(chv, s):
            pltpu.make_async_copy(x_hbm.at[pl.ds(tokb + chv * CH, CH)],
                                  xbuf.at[s], sem_in[s]).wait()

        def wait_sc(s):
            pltpu.make_async_copy(xbuf.at[s], xg_hbm.at[idxb.at[s]],
                                  sem_sc[s]).wait()

        start_in(0, 0)
        for chv in range(nch):
            s = chv % 2
            wait_in(chv, s)
            idxb.at[s][...] = posbuf[pl.ds(chv * CH, CH)]
            pltpu.async_copy(xbuf.at[s], xg_hbm.at[idxb.at[s]], sem_sc[s])
            if chv + 1 < nch:
                if chv >= 1:
                    wait_sc(1 - s)
                start_in(chv + 1, 1 - s)
        wait_sc((nch - 1) % 2)
        wait_sc(nch % 2)


def _dispatch(ep, x):
    mesh = plsc.VectorSubcoreMesh(core_axis_name="c", subcore_axis_name="s")
    i32 = jnp.int32
    f = pl.kernel(
        _dispatch_body,
        out_type=[
            jax.ShapeDtypeStruct((NP,), i32),
            jax.ShapeDtypeStruct((VMAX,), i32),
            jax.ShapeDtypeStruct((VMAX,), i32),
            jax.ShapeDtypeStruct((VMAX,), i32),
            jax.ShapeDtypeStruct((VMAX,), i32),
            jax.ShapeDtypeStruct((NP, H), jnp.float32),
        ],
        mesh=mesh,
        scratch_types=[
            pltpu.VMEM((NP,), i32),
            pltpu.VMEM((PPT2,), i32),
            pltpu.VMEM((PPT2,), i32),
            pltpu.VMEM((16,), i32),
            pltpu.VMEM((VMAX,), i32),
            pltpu.VMEM((VMAX,), i32),
            pltpu.VMEM((VMAX,), i32),
            pltpu.VMEM((VMAX,), i32),
            pltpu.VMEM((2, CH, H), jnp.float32),
            pltpu.VMEM((2, CH), i32),
            pltpu.SemaphoreType.DMA,
            pltpu.SemaphoreType.DMA,
            pltpu.SemaphoreType.DMA,
            pltpu.SemaphoreType.DMA,
        ],
        compiler_params=pltpu.CompilerParams(needs_layout_passes=False),
    )
    return f(ep, x)


def _visits_body(offs_ref, vb_ref, ve_ref, vr0_ref, vr1_ref):
    def mk_expert(e, v):
        o0 = offs_ref[e]
        o1 = offs_ref[e + 1]
        b0 = o0 // TBLK
        bend = (o1 + TBLK - 1) // TBLK

        def body(b, v):
            vb_ref[v] = b
            ve_ref[v] = e
            vr0_ref[v] = jnp.maximum(o0 - b * TBLK, 0)
            vr1_ref[v] = jnp.minimum(o1 - b * TBLK, TBLK)
            return v + 1

        return lax.fori_loop(b0, bend, body, v)

    v = 0
    for e in range(N_ROUTED):
        v = mk_expert(e, v)
    last_b = vb_ref[v - 1]
    last_e = ve_ref[v - 1]

    def pad(i, _):
        vb_ref[i] = last_b
        ve_ref[i] = last_e
        vr0_ref[i] = 0
        vr1_ref[i] = 0
        return 0

    lax.fori_loop(v, VMAX, pad, 0)


def _visits(offs):
    i32 = jnp.int32
    smem = functools.partial(pl.BlockSpec, memory_space=pltpu.SMEM)
    return pl.pallas_call(
        _visits_body,
        in_specs=[smem()],
        out_specs=[smem(), smem(), smem(), smem()],
        out_shape=[jax.ShapeDtypeStruct((VMAX,), i32)] * 4,
    )(offs)


def _xscatter_body(x_hbm, pos_hbm, xg_hbm, xbuf, idx1, idx2,
                   sem_in0, sem_in1, sem_sc0, sem_sc1):
    c = lax.axis_index("c")
    sid = lax.axis_index("s")
    wid = sid * 2 + c
    nch = TPT // CH
    sem_in = (sem_in0, sem_in1)
    sem_sc = (sem_sc0, sem_sc1)

    def start_in(chv, s):
        tb = wid * TPT + chv * CH
        pltpu.async_copy(x_hbm.at[pl.ds(tb, CH)], xbuf.at[s], sem_in[s])
        pltpu.async_copy(pos_hbm.at[pl.ds(tb, CH)], idx1.at[s], sem_in[s])
        pltpu.async_copy(pos_hbm.at[pl.ds(T + tb, CH)], idx2.at[s], sem_in[s])

    def wait_in(chv, s):
        tb = wid * TPT + chv * CH
        pltpu.make_async_copy(x_hbm.at[pl.ds(tb, CH)], xbuf.at[s],
                              sem_in[s]).wait()
        pltpu.make_async_copy(pos_hbm.at[pl.ds(tb, CH)], idx1.at[s],
                              sem_in[s]).wait()
        pltpu.make_async_copy(pos_hbm.at[pl.ds(T + tb, CH)], idx2.at[s],
                              sem_in[s]).wait()

    def start_sc(s):
        pltpu.async_copy(xbuf.at[s], xg_hbm.at[idx1.at[s]], sem_sc[s])
        pltpu.async_copy(xbuf.at[s], xg_hbm.at[idx2.at[s]], sem_sc[s])

    def wait_sc(s):
        pltpu.make_async_copy(xbuf.at[s], xg_hbm.at[idx1.at[s]],
                              sem_sc[s]).wait()
        pltpu.make_async_copy(xbuf.at[s], xg_hbm.at[idx2.at[s]],
                              sem_sc[s]).wait()

    start_in(0, 0)
    for chv in range(nch):
        s = chv % 2
        wait_in(chv, s)
        start_sc(s)
        if chv + 1 < nch:
            if chv >= 1:
                wait_sc(1 - s)
            start_in(chv + 1, 1 - s)
    wait_sc((nch - 1) % 2)
    wait_sc(nch % 2)


def _xscatter(x, pos):
    mesh = plsc.VectorSubcoreMesh(core_axis_name="c", subcore_axis_name="s")
    f = pl.kernel(
        _xscatter_body,
        out_type=jax.ShapeDtypeStruct((NP, H), jnp.float32),
        mesh=mesh,
        scratch_types=[
            pltpu.VMEM((2, CH, H), jnp.float32),
            pltpu.VMEM((2, CH), jnp.int32),
            pltpu.VMEM((2, CH), jnp.int32),
            pltpu.SemaphoreType.DMA,
            pltpu.SemaphoreType.DMA,
            pltpu.SemaphoreType.DMA,
            pltpu.SemaphoreType.DMA,
        ],
        compiler_params=pltpu.CompilerParams(needs_layout_passes=False),
    )
    return f(x, pos)


def _gmm_body(vb_ref, ve_ref, vr0_ref, vr1_ref, xg_ref, up_ref, down_ref,
              ys_ref):
    v = pl.program_id(0)
    r0 = vr0_ref[v]
    r1 = vr1_ref[v]

    @pl.when(r1 > r0)
    def _():
        xb = xg_ref[...].astype(jnp.bfloat16)
        h = lax.dot_general(xb, up_ref[0], (((1,), (1,)), ((), ())),
                            preferred_element_type=jnp.float32)
        h = h * 0.5 * (1.0 + lax.erf(h * 0.7071067811865476))
        y = lax.dot_general(h.astype(jnp.bfloat16), down_ref[0],
                            (((1,), (1,)), ((), ())),
                            preferred_element_type=jnp.float32)
        rows = lax.broadcasted_iota(jnp.int32, (TBLK, H), 0)
        keep = (rows >= r0) & (rows < r1)
        ys_ref[...] = jnp.where(keep, y, ys_ref[...])


def _gmm(vb, ve, vr0, vr1, xg, up_bf, down_bf):
    grid_spec = pltpu.PrefetchScalarGridSpec(
        num_scalar_prefetch=4,
        grid=(VMAX,),
        in_specs=[
            pl.BlockSpec((TBLK, H), lambda v, vb, ve, r0, r1: (vb[v], 0)),
            pl.BlockSpec((1, E_DIM, H), lambda v, vb, ve, r0, r1: (ve[v], 0, 0)),
            pl.BlockSpec((1, H, E_DIM), lambda v, vb, ve, r0, r1: (ve[v], 0, 0)),
        ],
        out_specs=pl.BlockSpec((TBLK, H), lambda v, vb, ve, r0, r1: (vb[v], 0)),
    )
    return pl.pallas_call(
        _gmm_body,
        grid_spec=grid_spec,
        out_shape=jax.ShapeDtypeStruct((NP, H), jnp.float32),
    )(vb, ve, vr0, vr1, xg, up_bf, down_bf)


def _shared_body(x_ref, u_ref, d_ref, w_ref, out_ref, eidx_ref, prob_ref):
    _router_body(x_ref, w_ref, eidx_ref, prob_ref)
    xb = x_ref[...].astype(jnp.bfloat16)
    h = lax.dot_general(xb, u_ref[...], (((1,), (1,)), ((), ())),
                        preferred_element_type=jnp.float32)
    h = h * 0.5 * (1.0 + lax.erf(h * 0.7071067811865476))
    out_ref[...] = lax.dot_general(h.astype(jnp.bfloat16), d_ref[...],
                                   (((1,), (1,)), ((), ())),
                                   preferred_element_type=jnp.float32)


def _shared_router(x, u_bf, d_bf, router_w):
    su = N_SHARED * E_DIM
    w_pad = jnp.zeros((H, LANES), jnp.float32).at[:, :N_ROUTED].set(router_w.T)
    return pl.pallas_call(
        _shared_body,
        grid=(T // TBLK,),
        in_specs=[
            pl.BlockSpec((TBLK, H), lambda t: (t, 0)),
            pl.BlockSpec((su, H), lambda t: (0, 0)),
            pl.BlockSpec((H, su), lambda t: (0, 0)),
            pl.BlockSpec((H, LANES), lambda t: (0, 0)),
        ],
        out_specs=[
            pl.BlockSpec((TBLK, H), lambda t: (t, 0)),
            pl.BlockSpec((TBLK, LANES), lambda t: (t, 0)),
            pl.BlockSpec((TBLK, LANES), lambda t: (t, 0)),
        ],
        out_shape=[
            jax.ShapeDtypeStruct((T, H), jnp.float32),
            jax.ShapeDtypeStruct((T, LANES), jnp.int32),
            jax.ShapeDtypeStruct((T, LANES), jnp.float32),
        ],
    )(x, u_bf, d_bf, w_pad)


CH5 = 8


def _combine_body(sh_hbm, ys_hbm, pos_hbm, wp_hbm, out_hbm,
                  sbuf, g1, g2, posb1, posb2, wb1, wb2,
                  sem_in0, sem_in1, sem_out0, sem_out1):
    c = lax.axis_index("c")
    sid = lax.axis_index("s")
    wid = sid * 2 + c
    tbase = wid * TPT
    nch = TPT // CH5
    sem_in = (sem_in0, sem_in1)
    sem_out = (sem_out0, sem_out1)

    pltpu.sync_copy(pos_hbm.at[pl.ds(tbase, TPT)], posb1)
    pltpu.sync_copy(pos_hbm.at[pl.ds(T + tbase, TPT)], posb2)
    pltpu.sync_copy(wp_hbm.at[pl.ds(tbase, TPT)], wb1.at[pl.ds(0, TPT)])
    pltpu.sync_copy(wp_hbm.at[pl.ds(T + tbase, TPT)], wb2.at[pl.ds(0, TPT)])

    def start_in(chv, s):
        tb = tbase + chv * CH5
        pltpu.async_copy(ys_hbm.at[posb1.at[pl.ds(chv * CH5, CH5)]],
                         g1.at[s], sem_in[s])
        pltpu.async_copy(ys_hbm.at[posb2.at[pl.ds(chv * CH5, CH5)]],
                         g2.at[s], sem_in[s])
        pltpu.async_copy(sh_hbm.at[pl.ds(tb, CH5)], sbuf.at[s], sem_in[s])

    def wait_in(chv, s):
        tb = tbase + chv * CH5
        pltpu.make_async_copy(ys_hbm.at[posb1.at[pl.ds(chv * CH5, CH5)]],
                              g1.at[s], sem_in[s]).wait()
        pltpu.make_async_copy(ys_hbm.at[posb2.at[pl.ds(chv * CH5, CH5)]],
                              g2.at[s], sem_in[s]).wait()
        pltpu.make_async_copy(sh_hbm.at[pl.ds(tb, CH5)], sbuf.at[s],
                              sem_in[s]).wait()

    def wait_out(chv, s):
        tb = tbase + chv * CH5
        pltpu.make_async_copy(sbuf.at[s], out_hbm.at[pl.ds(tb, CH5)],
                              sem_out[s]).wait()

    start_in(0, 0)
    for chv in range(nch):
        s = chv % 2
        if chv + 1 < nch:
            if chv >= 1:
                wait_out(chv - 1, 1 - s)
            start_in(chv + 1, 1 - s)
        wait_in(chv, s)
        wv1 = wb1[pl.ds(chv * CH5, 16)]
        wv2 = wb2[pl.ds(chv * CH5, 16)]
        iota16 = lax.iota(jnp.int32, 16)
        zf = jnp.zeros((16,), jnp.float32)

        def row(r, _):
            wa = jnp.sum(jnp.where(iota16 == r, wv1, zf))
            wb = jnp.sum(jnp.where(iota16 == r, wv2, zf))

            def col(j, _):
                for q in range(4):
                    cs = pl.ds(j * 64 + q * 16, 16)
                    sbuf.at[s][r, cs] = (sbuf.at[s][r, cs]
                                         + wa * g1.at[s][r, cs]
                                         + wb * g2.at[s][r, cs])
                return 0

            lax.fori_loop(0, H // 64, col, 0)
            return 0

        lax.fori_loop(0, CH5, row, 0)
        tb = tbase + chv * CH5
        pltpu.async_copy(sbuf.at[s], out_hbm.at[pl.ds(tb, CH5)], sem_out[s])
    wait_out(nch - 2, nch % 2)
    wait_out(nch - 1, (nch - 1) % 2)


def _combine(sh, ys, pos, wp):
    mesh = plsc.VectorSubcoreMesh(core_axis_name="c", subcore_axis_name="s")
    f32 = jnp.float32
    f = pl.kernel(
        _combine_body,
        out_type=jax.ShapeDtypeStruct((T, H), f32),
        mesh=mesh,
        scratch_types=[
            pltpu.VMEM((2, CH5, H), f32),
            pltpu.VMEM((2, CH5, H), f32),
            pltpu.VMEM((2, CH5, H), f32),
            pltpu.VMEM((TPT,), jnp.int32),
            pltpu.VMEM((TPT,), jnp.int32),
            pltpu.VMEM((TPT + 16,), f32),
            pltpu.VMEM((TPT + 16,), f32),
            pltpu.SemaphoreType.DMA,
            pltpu.SemaphoreType.DMA,
            pltpu.SemaphoreType.DMA,
            pltpu.SemaphoreType.DMA,
        ],
        compiler_params=pltpu.CompilerParams(needs_layout_passes=False),
    )
    return f(sh, ys, pos, wp)


def kernel(x, shared_up, shared_down, routed_up, routed_down, router_w):
    up_bf = routed_up.astype(jnp.bfloat16)
    down_bf = routed_down.astype(jnp.bfloat16)
    u_sh = shared_up.reshape(N_SHARED * E_DIM, H).astype(jnp.bfloat16)
    d_sh = jnp.concatenate([shared_down[i] for i in range(N_SHARED)],
                           axis=1).astype(jnp.bfloat16)

    sh, eidx, prob = _shared_router(x, u_sh, d_sh, router_w)
    ep = jnp.concatenate([eidx[:, 0], eidx[:, 1]])
    wp = jnp.concatenate([prob[:, 0], prob[:, 1]])

    pos, vb, ve, vr0, vr1, xg = _dispatch(ep, x)
    ys = _gmm(vb, ve, vr0, vr1, xg, up_bf, down_bf)
    out = _combine(sh, ys, pos, wp)
    return out
